# Initial kernel scaffold; baseline (speedup 1.0000x reference)
import jax
import jax.numpy as jnp
from jax.experimental import pallas as pl


def kernel(points, boxes, num_sample):
    raise NotImplementedError("write your pallas kernel here")


# speedup vs baseline: 3.9369x; 3.9369x over previous
import jax
import jax.numpy as jnp
from jax import lax
from jax.experimental import pallas as pl
from jax.experimental.pallas import tpu as pltpu
from jax.experimental.pallas import tpu_sc as plsc

N = 100000
B = 128
K = 32
F = 7
NW = 32
BPW = B // NW
CHUNK = 10000
NCH = N // CHUNK
SLICES = CHUNK // 16
IBUF_STRIDE = 64
GAMMA2 = 1.1 * 1.1


def _splat(buf, idx):
    return plsc.load_gather(buf, [jnp.full((16,), idx, jnp.int32)])


def _sc_body(xs_hbm, ys_hbm, pts8_hbm, boxes_hbm, out_hbm,
             xbuf, ybuf, bbuf, ibuf, gbuf, featbuf, outbuf, dsem):
    wid = lax.axis_index("c") * 16 + lax.axis_index("s")
    iota = lax.iota(jnp.int32, 16)

    pltpu.sync_copy(boxes_hbm, bbuf)
    base_box = wid * BPW
    cxs, cys, r2s, vxs, vys = [], [], [], [], []
    for j in range(BPW):
        o = (base_box + j) * 9
        cxs.append(_splat(bbuf, o + 0))
        cys.append(_splat(bbuf, o + 1))
        hx = _splat(bbuf, o + 3) * jnp.float32(0.5)
        hy = _splat(bbuf, o + 4) * jnp.float32(0.5)
        r2s.append((hx * hx + hy * hy) * jnp.float32(GAMMA2))
        vxs.append(_splat(bbuf, o + 7))
        vys.append(_splat(bbuf, o + 8))

    def chunk_body(c, cnts):
        off = pl.multiple_of(c * CHUNK, 8)
        pltpu.sync_copy(xs_hbm.at[pl.ds(off, CHUNK)], xbuf)
        pltpu.sync_copy(ys_hbm.at[pl.ds(off, CHUNK)], ybuf)

        def slice_body(s, cnts):
            xv = xbuf[pl.ds(s * 16, 16)]
            yv = ybuf[pl.ds(s * 16, 16)]
            pidxf = (c * CHUNK + s * 16 + iota).astype(jnp.float32)
            new = []
            for j in range(BPW):
                dx = xv - cxs[j]
                dy = yv - cys[j]
                m = dx * dx + dy * dy <= r2s[j]
                mi = m.astype(jnp.int32)
                woff = j * IBUF_STRIDE + jnp.minimum(cnts[j], K)
                pos = woff + plsc.cumsum(mi) - 1
                plsc.store_scatter(ibuf, [pos], pidxf, mask=m)
                new.append(cnts[j] + jnp.sum(mi))
            return tuple(new)

        return lax.fori_loop(0, SLICES, slice_body, cnts)

    zero = jnp.int32(0)
    cnts = lax.fori_loop(0, NCH, chunk_body, (zero,) * BPW)

    for j in range(BPW):
        cj = jnp.minimum(cnts[j], K)
        for t in (0, 16):
            v = ibuf[pl.ds(j * IBUF_STRIDE + t, 16)].astype(jnp.int32)
            valid = (t + iota) < cj
            gbuf[pl.ds(j * K + t, 16)] = jnp.where(valid, v, 0)
    pltpu.async_copy(pts8_hbm.at[gbuf], featbuf, dsem).wait()

    zf = jnp.zeros((16,), jnp.float32)

    def zero_body(i, _):
        outbuf[pl.ds(i * 16, 16)] = zf
        return 0

    lax.fori_loop(0, (BPW * K * F) // 16, zero_body, 0)

    for j in range(BPW):
        cj = jnp.minimum(cnts[j], K)
        for t in (0, 16):
            sl = j * K + t + iota
            valid = (t + iota) < cj
            base7 = sl * 7
            for f in range(5):
                col = jnp.full((16,), f, jnp.int32)
                val = plsc.load_gather(featbuf, [sl, col])
                plsc.store_scatter(outbuf, [base7 + f], val, mask=valid)
            plsc.store_scatter(outbuf, [base7 + 5], vxs[j], mask=valid)
            plsc.store_scatter(outbuf, [base7 + 6], vys[j], mask=valid)

    pltpu.sync_copy(outbuf, out_hbm.at[pl.ds(wid * (BPW * K * F), BPW * K * F)])


@jax.jit
def _voxel_sample(xs, ys, pts8, boxes_flat):
    mesh = plsc.VectorSubcoreMesh(core_axis_name="c", subcore_axis_name="s",
                                  num_cores=2, num_subcores=16)
    return pl.kernel(
        _sc_body,
        out_type=jax.ShapeDtypeStruct((B * K * F,), jnp.float32),
        mesh=mesh,
        compiler_params=pltpu.CompilerParams(needs_layout_passes=False,
                                             use_tc_tiling_on_sc=False),
        scratch_types=[
            pltpu.VMEM((CHUNK,), jnp.float32),
            pltpu.VMEM((CHUNK,), jnp.float32),
            pltpu.VMEM((B * 9 + 16,), jnp.float32),
            pltpu.VMEM((BPW * IBUF_STRIDE,), jnp.float32),
            pltpu.VMEM((BPW * K,), jnp.int32),
            pltpu.VMEM((BPW * K, 8), jnp.float32),
            pltpu.VMEM((BPW * K * F,), jnp.float32),
            pltpu.SemaphoreType.DMA,
        ],
    )(xs, ys, pts8, boxes_flat)


def kernel(points, boxes, num_sample):
    del num_sample
    xs = points[:, 0]
    ys = points[:, 1]
    pts8 = jnp.concatenate(
        [points, jnp.zeros((points.shape[0], 3), jnp.float32)], axis=1)
    boxes_flat = jnp.concatenate(
        [boxes.reshape(-1), jnp.zeros((16,), jnp.float32)])
    out = _voxel_sample(xs, ys, pts8, boxes_flat)
    return out.reshape(B, K, F)


# speedup vs baseline: 6.0739x; 1.5428x over previous
import jax
import jax.numpy as jnp
from jax import lax
from jax.experimental import pallas as pl
from jax.experimental.pallas import tpu as pltpu
from jax.experimental.pallas import tpu_sc as plsc

N = 100000
B = 128
K = 32
F = 7
NW = 32
BPW = B // NW
CHUNK = 10000
NCH = N // CHUNK
SLICES = CHUNK // 16
IBUF_STRIDE = 64
GAMMA2 = 1.1 * 1.1


def _splat(buf, idx):
    return plsc.load_gather(buf, [jnp.full((16,), idx, jnp.int32)])


def _sc_body(xs_hbm, ys_hbm, pts8_hbm, boxes_hbm, out_hbm,
             xbuf, ybuf, bbuf, ibuf, gbuf, featbuf, outbuf, dsem):
    wid = lax.axis_index("c") * 16 + lax.axis_index("s")
    iota = lax.iota(jnp.int32, 16)

    pltpu.sync_copy(boxes_hbm, bbuf)
    base_box = wid * BPW
    cxs, cys, r2s, vxs, vys = [], [], [], [], []
    for j in range(BPW):
        o = (base_box + j) * 9
        cxs.append(_splat(bbuf, o + 0))
        cys.append(_splat(bbuf, o + 1))
        hx = _splat(bbuf, o + 3) * jnp.float32(0.5)
        hy = _splat(bbuf, o + 4) * jnp.float32(0.5)
        r2s.append((hx * hx + hy * hy) * jnp.float32(GAMMA2))
        vxs.append(_splat(bbuf, o + 7))
        vys.append(_splat(bbuf, o + 8))

    kk = jnp.int32(K)

    def not_done(cnts):
        d = cnts[0] >= kk
        for j in range(1, BPW):
            d = d & (cnts[j] >= kk)
        return ~d

    def chunk_cond(st):
        return (st[0] < NCH) & not_done(st[1:])

    def chunk_body(st):
        c = st[0]
        off = pl.multiple_of(c * CHUNK, 8)
        pltpu.sync_copy(xs_hbm.at[pl.ds(off, CHUNK)], xbuf)
        pltpu.sync_copy(ys_hbm.at[pl.ds(off, CHUNK)], ybuf)

        def slice_cond(st):
            return (st[0] < SLICES) & not_done(st[1:])

        def slice_body(st):
            s = st[0]
            cnts = st[1:]
            xv = xbuf[pl.ds(s * 16, 16)]
            yv = ybuf[pl.ds(s * 16, 16)]
            pidxf = (c * CHUNK + s * 16 + iota).astype(jnp.float32)
            new = []
            for j in range(BPW):
                dx = xv - cxs[j]
                dy = yv - cys[j]
                m = dx * dx + dy * dy <= r2s[j]
                cs = plsc.cumsum(m.astype(jnp.int32))
                woff = j * IBUF_STRIDE + jnp.minimum(cnts[j], K)
                plsc.store_scatter(ibuf, [woff + cs - 1], pidxf, mask=m)
                new.append(cnts[j] + cs[15])
            return (s + 1,) + tuple(new)

        inner = lax.while_loop(slice_cond, slice_body, (jnp.int32(0),) + st[1:])
        return (c + 1,) + inner[1:]

    zero = jnp.int32(0)
    st = lax.while_loop(chunk_cond, chunk_body, (zero,) * (BPW + 1))
    cnts = st[1:]

    for j in range(BPW):
        cj = jnp.minimum(cnts[j], K)
        for t in (0, 16):
            v = ibuf[pl.ds(j * IBUF_STRIDE + t, 16)].astype(jnp.int32)
            valid = (t + iota) < cj
            gbuf[pl.ds(j * K + t, 16)] = jnp.where(valid, v, 0)
    pltpu.async_copy(pts8_hbm.at[gbuf], featbuf, dsem).wait()

    zf = jnp.zeros((16,), jnp.float32)

    def zero_body(i, _):
        outbuf[pl.ds(i * 16, 16)] = zf
        return 0

    lax.fori_loop(0, (BPW * K * F) // 16, zero_body, 0)

    for j in range(BPW):
        cj = jnp.minimum(cnts[j], K)
        for t in (0, 16):
            sl = j * K + t + iota
            valid = (t + iota) < cj
            base7 = sl * 7
            for f in range(5):
                col = jnp.full((16,), f, jnp.int32)
                val = plsc.load_gather(featbuf, [sl, col])
                plsc.store_scatter(outbuf, [base7 + f], val, mask=valid)
            plsc.store_scatter(outbuf, [base7 + 5], vxs[j], mask=valid)
            plsc.store_scatter(outbuf, [base7 + 6], vys[j], mask=valid)

    pltpu.sync_copy(outbuf, out_hbm.at[pl.ds(wid * (BPW * K * F), BPW * K * F)])


@jax.jit
def _voxel_sample(xs, ys, pts8, boxes_flat):
    mesh = plsc.VectorSubcoreMesh(core_axis_name="c", subcore_axis_name="s",
                                  num_cores=2, num_subcores=16)
    return pl.kernel(
        _sc_body,
        out_type=jax.ShapeDtypeStruct((B * K * F,), jnp.float32),
        mesh=mesh,
        compiler_params=pltpu.CompilerParams(needs_layout_passes=False,
                                             use_tc_tiling_on_sc=False),
        scratch_types=[
            pltpu.VMEM((CHUNK,), jnp.float32),
            pltpu.VMEM((CHUNK,), jnp.float32),
            pltpu.VMEM((B * 9 + 16,), jnp.float32),
            pltpu.VMEM((BPW * IBUF_STRIDE,), jnp.float32),
            pltpu.VMEM((BPW * K,), jnp.int32),
            pltpu.VMEM((BPW * K, 8), jnp.float32),
            pltpu.VMEM((BPW * K * F,), jnp.float32),
            pltpu.SemaphoreType.DMA,
        ],
    )(xs, ys, pts8, boxes_flat)


def kernel(points, boxes, num_sample):
    del num_sample
    xs = points[:, 0]
    ys = points[:, 1]
    pts8 = jnp.concatenate(
        [points, jnp.zeros((points.shape[0], 3), jnp.float32)], axis=1)
    boxes_flat = jnp.concatenate(
        [boxes.reshape(-1), jnp.zeros((16,), jnp.float32)])
    out = _voxel_sample(xs, ys, pts8, boxes_flat)
    return out.reshape(B, K, F)
